# baseline (speedup 1.0000x reference)
import functools

import jax
import jax.numpy as jnp
from jax import lax
from jax.experimental import pallas as pl
from jax.experimental.pallas import tpu as pltpu
from jax.experimental.pallas import tpu_sc as plsc

_EPS = 1e-5


def _sc_scatter_body(b_per_w, nc, table_hbm, pos_hbm, out_hbm, idx_v, rows_v, sem):
    wid = lax.axis_index("s") * nc + lax.axis_index("c")
    base = wid * b_per_w
    pltpu.sync_copy(pos_hbm.at[pl.ds(base, b_per_w)], idx_v)
    pltpu.sync_copy(table_hbm.at[pl.ds(base, b_per_w)], rows_v)
    pltpu.async_copy(rows_v, out_hbm.at[idx_v], sem).wait()


def _sc_scatter(table, pos):
    N, D = table.shape
    info = plsc.get_sparse_core_info()
    nc, ns = info.num_cores, info.num_subcores
    b_per_w = N // (nc * ns)
    mesh = plsc.VectorSubcoreMesh(core_axis_name="c", subcore_axis_name="s")
    return pl.kernel(
        functools.partial(_sc_scatter_body, b_per_w, nc),
        mesh=mesh,
        out_type=jax.ShapeDtypeStruct((N, D), jnp.float32),
        scratch_types=[
            pltpu.VMEM((b_per_w,), jnp.int32),
            pltpu.VMEM((b_per_w, D), jnp.float32),
            pltpu.SemaphoreType.DMA,
        ],
    )(table, pos)


def _ln_rows(x, g, b):
    mu = jnp.mean(x, axis=-1, keepdims=True)
    var = jnp.mean((x - mu) ** 2, axis=-1, keepdims=True)
    return (x - mu) * jax.lax.rsqrt(var + _EPS) * g + b


def _mega_kernel(n_cls, T, nt,
                 tok_ref, lab_ref, dic_ref, g_ref, b_ref, o_ref,
                 ln_ref, seg_ref, cnt_ref, posT_ref, dpos_ref):
    p = pl.program_id(0)

    @pl.when(p == 0)
    def _init():
        seg_ref[...] = jnp.zeros_like(seg_ref)
        cnt_ref[...] = jnp.zeros_like(cnt_ref)
        posT_ref[0, 0] = 0.0
        dpos_ref[0, 0] = 0.0

    @pl.when(p < nt)
    def _band_phase():
        i = p
        row = _ln_rows(tok_ref[pl.ds(i * T, T), :], g_ref[...], b_ref[...])
        ln_ref[pl.ds(i * T, T), :] = row
        labr = lab_ref[i, 0, :]

        cls = jax.lax.broadcasted_iota(jnp.int32, (n_cls, T), 0)
        onehot = (cls == labr[None, :]).astype(jnp.float32)
        seg_ref[...] += jax.lax.dot_general(
            onehot, row, (((1,), (0,)), ((), ())),
            preferred_element_type=jnp.float32,
        )
        cnt_ref[...] += jnp.sum(onehot, axis=1)[None, :]
        g = jax.lax.dot_general(
            dic_ref[...], row, (((1,), (1,)), ((), ())),
            preferred_element_type=jnp.float32,
        ) * (1.0 / 768.0)
        dpos_ref[0, 0] += 2.0 * jnp.sum(jnp.where(onehot > 0.0, jnp.exp(g), 0.0))

        lab_all = lab_ref[...]
        row_first = jnp.min(labr)
        jlo = jnp.sum((lab_all < row_first).astype(jnp.int32)) // T

        def body(j, acc):
            col = ln_ref[pl.ds(j * T, T), :]
            s = jax.lax.dot_general(
                row, col, (((1,), (1,)), ((), ())),
                preferred_element_type=jnp.float32,
            ) * (1.0 / 768.0)
            labc = lab_ref[j, 0, :]
            m = labr[:, None] == labc[None, :]
            w = jnp.where(j == i, 1.0, 2.0)
            return acc + w * jnp.sum(jnp.where(m, jnp.exp(s), 0.0))

        acc = jax.lax.fori_loop(jlo, i + 1, body, jnp.float32(0.0))
        posT_ref[0, 0] += acc

    @pl.when(p == nt)
    def _final_phase():
        dic = dic_ref[...]
        diag = jnp.sum(dic * dic, axis=1) * (1.0 / 768.0)
        pos_dic_diag = jnp.sum(jnp.exp(diag))

        cnt = cnt_ref[0, :]
        char_tem = (dic + seg_ref[...]) / (1.0 + cnt)[:, None]
        updated = dic + 0.1 * char_tem
        rowi = jax.lax.broadcasted_iota(jnp.int32, (n_cls, 1), 0)
        new_dic = jnp.where(rowi == 0, dic, updated)
        nd = _ln_rows(new_dic, g_ref[...], b_ref[...])

        sim = jax.lax.dot_general(
            nd, nd, (((1,), (1,)), ((), ())),
            preferred_element_type=jnp.float32,
        ) * (1.0 / 768.0)
        rr = jax.lax.broadcasted_iota(jnp.int32, (n_cls, n_cls), 0)
        cc = jax.lax.broadcasted_iota(jnp.int32, (n_cls, n_cls), 1)
        keep = jnp.logical_and(rr > 0, cc > 0)
        neg = jnp.sum(jnp.where(keep, jnp.exp(sim), 0.0))

        pos = posT_ref[0, 0] + dpos_ref[0, 0] + pos_dic_diag
        o_ref[...] = (-jnp.log(pos / neg)).reshape(1, 1)


def kernel(input_f, target, char_dic, ln1_gamma, ln1_beta):
    B, L, D = input_f.shape
    N = B * L
    n_cls = char_dic.shape[0]
    tokens = input_f.reshape(N, D)
    labels = target.reshape(N).astype(jnp.int32)
    g2 = ln1_gamma.reshape(1, D)
    b2 = ln1_beta.reshape(1, D)

    cls1 = jnp.arange(n_cls, dtype=jnp.int32)
    onehot = (labels[None, :] == cls1[:, None]).astype(jnp.int32)
    csum = jnp.cumsum(onehot, axis=1)
    counts = csum[:, -1]
    cum_incl = jnp.cumsum(counts)
    offs = cum_incl - counts
    rank = jnp.sum(onehot * (csum - 1), axis=0)
    pos = jnp.sum(onehot * offs[:, None], axis=0) + rank
    slab = jnp.sum(
        (jnp.arange(N, dtype=jnp.int32)[:, None] >= cum_incl[None, :]).astype(jnp.int32),
        axis=1,
    )
    stok = _sc_scatter(tokens, pos)

    T = 256
    nt = N // T
    lab3 = slab.reshape(nt, 1, T)

    loss = pl.pallas_call(
        functools.partial(_mega_kernel, n_cls, T, nt),
        grid=(nt + 1,),
        in_specs=[
            pl.BlockSpec((N, D), lambda p: (0, 0)),
            pl.BlockSpec((nt, 1, T), lambda p: (0, 0, 0)),
            pl.BlockSpec((n_cls, D), lambda p: (0, 0)),
            pl.BlockSpec((1, D), lambda p: (0, 0)),
            pl.BlockSpec((1, D), lambda p: (0, 0)),
        ],
        out_specs=pl.BlockSpec((1, 1), lambda p: (0, 0)),
        out_shape=jax.ShapeDtypeStruct((1, 1), jnp.float32),
        scratch_shapes=[
            pltpu.VMEM((N, D), jnp.float32),
            pltpu.VMEM((n_cls, D), jnp.float32),
            pltpu.VMEM((1, n_cls), jnp.float32),
            pltpu.SMEM((1, 1), jnp.float32),
            pltpu.SMEM((1, 1), jnp.float32),
        ],
    )(stok, lab3, char_dic, g2, b2)

    return loss.reshape(1)


# speedup vs baseline: 1.1026x; 1.1026x over previous
import functools

import jax
import jax.numpy as jnp
from jax import lax
from jax.experimental import pallas as pl
from jax.experimental.pallas import tpu as pltpu
from jax.experimental.pallas import tpu_sc as plsc

_EPS = 1e-5


def _sc_gather_body(b_per_w, nc, table_hbm, idx_hbm, out_hbm, idx_v, rows_v, sem):
    wid = lax.axis_index("s") * nc + lax.axis_index("c")
    base = wid * b_per_w
    pltpu.sync_copy(idx_hbm.at[pl.ds(base, b_per_w)], idx_v)
    pltpu.async_copy(table_hbm.at[idx_v], rows_v, sem).wait()
    pltpu.sync_copy(rows_v, out_hbm.at[pl.ds(base, b_per_w)])


def _sc_gather(table, idx):
    N, D = table.shape
    info = plsc.get_sparse_core_info()
    nc, ns = info.num_cores, info.num_subcores
    b_per_w = N // (nc * ns)
    mesh = plsc.VectorSubcoreMesh(core_axis_name="c", subcore_axis_name="s")
    return pl.kernel(
        functools.partial(_sc_gather_body, b_per_w, nc),
        mesh=mesh,
        out_type=jax.ShapeDtypeStruct((N, D), jnp.float32),
        scratch_types=[
            pltpu.VMEM((b_per_w,), jnp.int32),
            pltpu.VMEM((b_per_w, D), jnp.float32),
            pltpu.SemaphoreType.DMA,
        ],
    )(table, idx)


def _ln_rows(x, g, b):
    mu = jnp.mean(x, axis=-1, keepdims=True)
    var = jnp.mean((x - mu) ** 2, axis=-1, keepdims=True)
    return (x - mu) * jax.lax.rsqrt(var + _EPS) * g + b


def _mega_kernel(n_cls, T, nt,
                 tok_ref, lab_ref, dic_ref, g_ref, b_ref, o_ref,
                 ln_ref, seg_ref, cnt_ref, posT_ref, dpos_ref):
    p = pl.program_id(0)

    @pl.when(p == 0)
    def _init():
        seg_ref[...] = jnp.zeros_like(seg_ref)
        cnt_ref[...] = jnp.zeros_like(cnt_ref)
        posT_ref[0, 0] = 0.0
        dpos_ref[0, 0] = 0.0

    @pl.when(p < nt)
    def _band_phase():
        i = p
        row = _ln_rows(tok_ref[pl.ds(i * T, T), :], g_ref[...], b_ref[...])
        ln_ref[pl.ds(i * T, T), :] = row
        labr = lab_ref[i, 0, :]

        cls = jax.lax.broadcasted_iota(jnp.int32, (n_cls, T), 0)
        onehot = (cls == labr[None, :]).astype(jnp.float32)
        seg_ref[...] += jax.lax.dot_general(
            onehot, row, (((1,), (0,)), ((), ())),
            preferred_element_type=jnp.float32,
        )
        cnt_ref[...] += jnp.sum(onehot, axis=1)[None, :]
        g = jax.lax.dot_general(
            dic_ref[...], row, (((1,), (1,)), ((), ())),
            preferred_element_type=jnp.float32,
        ) * (1.0 / 768.0)
        dpos_ref[0, 0] += 2.0 * jnp.sum(jnp.where(onehot > 0.0, jnp.exp(g), 0.0))

        lab_all = lab_ref[...]
        row_first = jnp.min(labr)
        jlo = jnp.sum((lab_all < row_first).astype(jnp.int32)) // T

        def body(j, acc):
            col = ln_ref[pl.ds(j * T, T), :]
            s = jax.lax.dot_general(
                row, col, (((1,), (1,)), ((), ())),
                preferred_element_type=jnp.float32,
            ) * (1.0 / 768.0)
            labc = lab_ref[j, 0, :]
            m = labr[:, None] == labc[None, :]
            w = jnp.where(j == i, 1.0, 2.0)
            return acc + w * jnp.sum(jnp.where(m, jnp.exp(s), 0.0))

        acc = jax.lax.fori_loop(jlo, i + 1, body, jnp.float32(0.0))
        posT_ref[0, 0] += acc

    @pl.when(p == nt)
    def _final_phase():
        dic = dic_ref[...]
        diag = jnp.sum(dic * dic, axis=1) * (1.0 / 768.0)
        pos_dic_diag = jnp.sum(jnp.exp(diag))

        cnt = cnt_ref[0, :]
        char_tem = (dic + seg_ref[...]) / (1.0 + cnt)[:, None]
        updated = dic + 0.1 * char_tem
        rowi = jax.lax.broadcasted_iota(jnp.int32, (n_cls, 1), 0)
        new_dic = jnp.where(rowi == 0, dic, updated)
        nd = _ln_rows(new_dic, g_ref[...], b_ref[...])

        sim = jax.lax.dot_general(
            nd, nd, (((1,), (1,)), ((), ())),
            preferred_element_type=jnp.float32,
        ) * (1.0 / 768.0)
        rr = jax.lax.broadcasted_iota(jnp.int32, (n_cls, n_cls), 0)
        cc = jax.lax.broadcasted_iota(jnp.int32, (n_cls, n_cls), 1)
        keep = jnp.logical_and(rr > 0, cc > 0)
        neg = jnp.sum(jnp.where(keep, jnp.exp(sim), 0.0))

        pos = posT_ref[0, 0] + dpos_ref[0, 0] + pos_dic_diag
        o_ref[...] = (-jnp.log(pos / neg)).reshape(1, 1)


def kernel(input_f, target, char_dic, ln1_gamma, ln1_beta):
    B, L, D = input_f.shape
    N = B * L
    n_cls = char_dic.shape[0]
    tokens = input_f.reshape(N, D)
    labels = target.reshape(N).astype(jnp.int32)
    g2 = ln1_gamma.reshape(1, D)
    b2 = ln1_beta.reshape(1, D)

    idx = jnp.arange(N, dtype=jnp.int32)
    skey = jnp.sort(labels * N + idx)
    perm = jnp.bitwise_and(skey, N - 1)
    slab = jax.lax.shift_right_logical(skey, 12)
    stok = _sc_gather(tokens, perm)

    T = 128
    nt = N // T
    lab3 = slab.reshape(nt, 1, T)

    loss = pl.pallas_call(
        functools.partial(_mega_kernel, n_cls, T, nt),
        grid=(nt + 1,),
        in_specs=[
            pl.BlockSpec((N, D), lambda p: (0, 0)),
            pl.BlockSpec((nt, 1, T), lambda p: (0, 0, 0)),
            pl.BlockSpec((n_cls, D), lambda p: (0, 0)),
            pl.BlockSpec((1, D), lambda p: (0, 0)),
            pl.BlockSpec((1, D), lambda p: (0, 0)),
        ],
        out_specs=pl.BlockSpec((1, 1), lambda p: (0, 0)),
        out_shape=jax.ShapeDtypeStruct((1, 1), jnp.float32),
        scratch_shapes=[
            pltpu.VMEM((N, D), jnp.float32),
            pltpu.VMEM((n_cls, D), jnp.float32),
            pltpu.VMEM((1, n_cls), jnp.float32),
            pltpu.SMEM((1, 1), jnp.float32),
            pltpu.SMEM((1, 1), jnp.float32),
        ],
    )(stok, lab3, char_dic, g2, b2)

    return loss.reshape(1)


# speedup vs baseline: 1.3201x; 1.1973x over previous
import functools

import jax
import jax.numpy as jnp
from jax import lax
from jax.experimental import pallas as pl
from jax.experimental.pallas import tpu as pltpu
from jax.experimental.pallas import tpu_sc as plsc

_EPS = 1e-5


def _sc_gather_body(b_per_w, nc, table_hbm, idx_hbm, out_hbm, idx_v, rows_v, sem):
    wid = lax.axis_index("s") * nc + lax.axis_index("c")
    base = wid * b_per_w
    pltpu.sync_copy(idx_hbm.at[pl.ds(base, b_per_w)], idx_v)
    pltpu.async_copy(table_hbm.at[idx_v], rows_v, sem).wait()
    pltpu.sync_copy(rows_v, out_hbm.at[pl.ds(base, b_per_w)])


def _sc_gather(table, idx):
    M = idx.shape[0]
    D = table.shape[1]
    info = plsc.get_sparse_core_info()
    nc, ns = info.num_cores, info.num_subcores
    b_per_w = M // (nc * ns)
    mesh = plsc.VectorSubcoreMesh(core_axis_name="c", subcore_axis_name="s")
    return pl.kernel(
        functools.partial(_sc_gather_body, b_per_w, nc),
        mesh=mesh,
        out_type=jax.ShapeDtypeStruct((M, D), jnp.float32),
        scratch_types=[
            pltpu.VMEM((b_per_w,), jnp.int32),
            pltpu.VMEM((b_per_w, D), jnp.float32),
            pltpu.SemaphoreType.DMA,
        ],
    )(table, idx)


def _ln_rows(x, g, b):
    mu = jnp.mean(x, axis=-1, keepdims=True)
    var = jnp.mean((x - mu) ** 2, axis=-1, keepdims=True)
    return (x - mu) * jax.lax.rsqrt(var + _EPS) * g + b


def _band_step(T, i, row, labr, lab_ref, ln_ref, dic_ref, seg_ref, cnt_ref,
               n_cls):
    cls = jax.lax.broadcasted_iota(jnp.int32, (n_cls, T), 0)
    onehot = (cls == labr[None, :]).astype(jnp.float32)
    seg_ref[...] += jax.lax.dot_general(
        onehot, row, (((1,), (0,)), ((), ())),
        preferred_element_type=jnp.float32,
    )
    cnt_ref[...] += jnp.sum(onehot, axis=1)[None, :]
    g = jax.lax.dot_general(
        dic_ref[...], row, (((1,), (1,)), ((), ())),
        preferred_element_type=jnp.float32,
    ) * (1.0 / 768.0)
    dpos = 2.0 * jnp.sum(jnp.where(onehot > 0.0, jnp.exp(g), 0.0))

    lab_all = lab_ref[...]
    row_first = jnp.min(labr)
    jlo = jnp.sum((lab_all < row_first).astype(jnp.int32)) // T

    def body(j, acc):
        col = ln_ref[pl.ds(j * T, T), :]
        s = jax.lax.dot_general(
            row, col, (((1,), (1,)), ((), ())),
            preferred_element_type=jnp.float32,
        ) * (1.0 / 768.0)
        labc = lab_ref[j, 0, :]
        m = labr[:, None] == labc[None, :]
        w = jnp.where(j == i, 1.0, 2.0)
        return acc + w * jnp.sum(jnp.where(m, jnp.exp(s), 0.0))

    return dpos + jax.lax.fori_loop(jlo, i + 1, body, jnp.float32(0.0))


def _half_a_kernel(n_cls, T,
                   tok_ref, lab_ref, dic_ref, g_ref, b_ref,
                   ln_ref, seg_ref, cnt_ref, pos_ref):
    p = pl.program_id(0)

    @pl.when(p == 0)
    def _init():
        seg_ref[...] = jnp.zeros_like(seg_ref)
        cnt_ref[...] = jnp.zeros_like(cnt_ref)
        pos_ref[...] = jnp.zeros_like(pos_ref)

    row = _ln_rows(tok_ref[pl.ds(p * T, T), :], g_ref[...], b_ref[...])
    ln_ref[pl.ds(p * T, T), :] = row
    labr = lab_ref[p, 0, :]
    acc = _band_step(T, p, row, labr, lab_ref, ln_ref, dic_ref,
                     seg_ref, cnt_ref, n_cls)
    pos_ref[...] = pos_ref[...] + acc.reshape(1, 1)


def _half_b_kernel(n_cls, T, nth, nt,
                   tok_ref, lab_ref, dic_ref, g_ref, b_ref,
                   ln1_ref, segA_ref, cntA_ref, posA_ref, o_ref,
                   ln_ref, seg_ref, cnt_ref, posT_ref):
    p = pl.program_id(0)

    @pl.when(p == 0)
    def _init():
        ln_ref[pl.ds(0, nth * T), :] = ln1_ref[...]
        seg_ref[...] = segA_ref[...]
        cnt_ref[...] = cntA_ref[...]
        posT_ref[0, 0] = posA_ref[0, 0]

    @pl.when(p < nth)
    def _band_phase():
        i = nth + p
        row = _ln_rows(tok_ref[pl.ds(p * T, T), :], g_ref[...], b_ref[...])
        ln_ref[pl.ds(i * T, T), :] = row
        labr = lab_ref[i, 0, :]
        acc = _band_step(T, i, row, labr, lab_ref, ln_ref, dic_ref,
                         seg_ref, cnt_ref, n_cls)
        posT_ref[0, 0] += acc

    @pl.when(p == nth)
    def _final_phase():
        dic = dic_ref[...]
        diag = jnp.sum(dic * dic, axis=1) * (1.0 / 768.0)
        pos_dic_diag = jnp.sum(jnp.exp(diag))

        cnt = cnt_ref[0, :]
        char_tem = (dic + seg_ref[...]) / (1.0 + cnt)[:, None]
        updated = dic + 0.1 * char_tem
        rowi = jax.lax.broadcasted_iota(jnp.int32, (n_cls, 1), 0)
        new_dic = jnp.where(rowi == 0, dic, updated)
        nd = _ln_rows(new_dic, g_ref[...], b_ref[...])

        sim = jax.lax.dot_general(
            nd, nd, (((1,), (1,)), ((), ())),
            preferred_element_type=jnp.float32,
        ) * (1.0 / 768.0)
        rr = jax.lax.broadcasted_iota(jnp.int32, (n_cls, n_cls), 0)
        cc = jax.lax.broadcasted_iota(jnp.int32, (n_cls, n_cls), 1)
        keep = jnp.logical_and(rr > 0, cc > 0)
        neg = jnp.sum(jnp.where(keep, jnp.exp(sim), 0.0))

        pos = posT_ref[0, 0] + pos_dic_diag
        o_ref[...] = (-jnp.log(pos / neg)).reshape(1, 1)


def kernel(input_f, target, char_dic, ln1_gamma, ln1_beta):
    B, L, D = input_f.shape
    N = B * L
    n_cls = char_dic.shape[0]
    tokens = input_f.reshape(N, D)
    labels = target.reshape(N).astype(jnp.int32)
    g2 = ln1_gamma.reshape(1, D)
    b2 = ln1_beta.reshape(1, D)

    idx = jnp.arange(N, dtype=jnp.int32)
    skey = jnp.sort(labels * N + idx)
    perm = jnp.bitwise_and(skey, N - 1)
    slab = jax.lax.shift_right_logical(skey, 12)

    T = 256
    nt = N // T
    nth = nt // 2
    H = N // 2
    lab3 = slab.reshape(nt, 1, T)

    stok1 = _sc_gather(tokens, perm[:H])
    stok2 = _sc_gather(tokens, perm[H:])

    ln1, segA, cntA, posA = pl.pallas_call(
        functools.partial(_half_a_kernel, n_cls, T),
        grid=(nth,),
        in_specs=[
            pl.BlockSpec((H, D), lambda p: (0, 0)),
            pl.BlockSpec((nth, 1, T), lambda p: (0, 0, 0)),
            pl.BlockSpec((n_cls, D), lambda p: (0, 0)),
            pl.BlockSpec((1, D), lambda p: (0, 0)),
            pl.BlockSpec((1, D), lambda p: (0, 0)),
        ],
        out_specs=[
            pl.BlockSpec((H, D), lambda p: (0, 0)),
            pl.BlockSpec((n_cls, D), lambda p: (0, 0)),
            pl.BlockSpec((1, n_cls), lambda p: (0, 0)),
            pl.BlockSpec((1, 1), lambda p: (0, 0)),
        ],
        out_shape=[
            jax.ShapeDtypeStruct((H, D), jnp.float32),
            jax.ShapeDtypeStruct((n_cls, D), jnp.float32),
            jax.ShapeDtypeStruct((1, n_cls), jnp.float32),
            jax.ShapeDtypeStruct((1, 1), jnp.float32),
        ],
    )(stok1, lab3[:nth], char_dic, g2, b2)

    loss = pl.pallas_call(
        functools.partial(_half_b_kernel, n_cls, T, nth, nt),
        grid=(nth + 1,),
        in_specs=[
            pl.BlockSpec((H, D), lambda p: (0, 0)),
            pl.BlockSpec((nt, 1, T), lambda p: (0, 0, 0)),
            pl.BlockSpec((n_cls, D), lambda p: (0, 0)),
            pl.BlockSpec((1, D), lambda p: (0, 0)),
            pl.BlockSpec((1, D), lambda p: (0, 0)),
            pl.BlockSpec((H, D), lambda p: (0, 0)),
            pl.BlockSpec((n_cls, D), lambda p: (0, 0)),
            pl.BlockSpec((1, n_cls), lambda p: (0, 0)),
            pl.BlockSpec((1, 1), lambda p: (0, 0)),
        ],
        out_specs=pl.BlockSpec((1, 1), lambda p: (0, 0)),
        out_shape=jax.ShapeDtypeStruct((1, 1), jnp.float32),
        scratch_shapes=[
            pltpu.VMEM((N, D), jnp.float32),
            pltpu.VMEM((n_cls, D), jnp.float32),
            pltpu.VMEM((1, n_cls), jnp.float32),
            pltpu.SMEM((1, 1), jnp.float32),
        ],
    )(stok2, lab3, char_dic, g2, b2, ln1, segA, cntA, posA)

    return loss.reshape(1)


# speedup vs baseline: 1.3507x; 1.0232x over previous
import functools

import jax
import jax.numpy as jnp
from jax import lax
from jax.experimental import pallas as pl
from jax.experimental.pallas import tpu as pltpu
from jax.experimental.pallas import tpu_sc as plsc

_EPS = 1e-5


def _sc_gather_body(b_per_w, nc, table_hbm, idx_hbm, out_hbm, idx_v, rows_v, sem):
    wid = lax.axis_index("s") * nc + lax.axis_index("c")
    base = wid * b_per_w
    pltpu.sync_copy(idx_hbm.at[pl.ds(base, b_per_w)], idx_v)
    pltpu.async_copy(table_hbm.at[idx_v], rows_v, sem).wait()
    pltpu.sync_copy(rows_v, out_hbm.at[pl.ds(base, b_per_w)])


def _sc_gather(table, idx):
    N, D = table.shape
    info = plsc.get_sparse_core_info()
    nc, ns = info.num_cores, info.num_subcores
    b_per_w = N // (nc * ns)
    mesh = plsc.VectorSubcoreMesh(core_axis_name="c", subcore_axis_name="s")
    return pl.kernel(
        functools.partial(_sc_gather_body, b_per_w, nc),
        mesh=mesh,
        out_type=jax.ShapeDtypeStruct((N, D), jnp.float32),
        scratch_types=[
            pltpu.VMEM((b_per_w,), jnp.int32),
            pltpu.VMEM((b_per_w, D), jnp.float32),
            pltpu.SemaphoreType.DMA,
        ],
    )(table, idx)


def _ln_rows(x, g, b):
    mu = jnp.mean(x, axis=-1, keepdims=True)
    var = jnp.mean((x - mu) ** 2, axis=-1, keepdims=True)
    return (x - mu) * jax.lax.rsqrt(var + _EPS) * g + b


def _mega_kernel(n_cls, T, nt,
                 tok_ref, lab_ref, dic_ref, g_ref, b_ref, o_ref,
                 ln_ref, seg_ref, cnt_ref, posT_ref, dpos_ref):
    p = pl.program_id(0)

    @pl.when(p == 0)
    def _init():
        seg_ref[...] = jnp.zeros_like(seg_ref)
        cnt_ref[...] = jnp.zeros_like(cnt_ref)
        posT_ref[0, 0] = 0.0
        dpos_ref[0, 0] = 0.0

    @pl.when(p < nt)
    def _band_phase():
        i = p
        row = _ln_rows(tok_ref[pl.ds(i * T, T), :], g_ref[...], b_ref[...])
        ln_ref[pl.ds(i * T, T), :] = row
        labr = lab_ref[i, 0, :]

        cls = jax.lax.broadcasted_iota(jnp.int32, (n_cls, T), 0)
        onehot = (cls == labr[None, :]).astype(jnp.float32)
        seg_ref[...] += jax.lax.dot_general(
            onehot, row, (((1,), (0,)), ((), ())),
            preferred_element_type=jnp.float32,
        )
        cnt_ref[...] += jnp.sum(onehot, axis=1)[None, :]
        sel = jax.lax.dot_general(
            onehot, dic_ref[...], (((0,), (0,)), ((), ())),
            preferred_element_type=jnp.float32,
        )
        d = jnp.sum(row * sel, axis=1) * (1.0 / 768.0)
        dpos_ref[0, 0] += 2.0 * jnp.sum(jnp.exp(d))

        lab_all = lab_ref[...]
        row_first = jnp.min(labr)
        jlo = jnp.sum((lab_all < row_first).astype(jnp.int32)) // T

        def body(j, acc):
            col = ln_ref[pl.ds(j * T, T), :]
            s = jax.lax.dot_general(
                row, col, (((1,), (1,)), ((), ())),
                preferred_element_type=jnp.float32,
            ) * (1.0 / 768.0)
            labc = lab_ref[j, 0, :]
            m = labr[:, None] == labc[None, :]
            w = jnp.where(j == i, 1.0, 2.0)
            return acc + w * jnp.sum(jnp.where(m, jnp.exp(s), 0.0))

        acc = jax.lax.fori_loop(jlo, i + 1, body, jnp.float32(0.0))
        posT_ref[0, 0] += acc

    @pl.when(p == nt)
    def _final_phase():
        dic = dic_ref[...]
        diag = jnp.sum(dic * dic, axis=1) * (1.0 / 768.0)
        pos_dic_diag = jnp.sum(jnp.exp(diag))

        cnt = cnt_ref[0, :]
        char_tem = (dic + seg_ref[...]) / (1.0 + cnt)[:, None]
        updated = dic + 0.1 * char_tem
        rowi = jax.lax.broadcasted_iota(jnp.int32, (n_cls, 1), 0)
        new_dic = jnp.where(rowi == 0, dic, updated)
        nd = _ln_rows(new_dic, g_ref[...], b_ref[...])

        sim = jax.lax.dot_general(
            nd, nd, (((1,), (1,)), ((), ())),
            preferred_element_type=jnp.float32,
        ) * (1.0 / 768.0)
        rr = jax.lax.broadcasted_iota(jnp.int32, (n_cls, n_cls), 0)
        cc = jax.lax.broadcasted_iota(jnp.int32, (n_cls, n_cls), 1)
        keep = jnp.logical_and(rr > 0, cc > 0)
        neg = jnp.sum(jnp.where(keep, jnp.exp(sim), 0.0))

        pos = posT_ref[0, 0] + dpos_ref[0, 0] + pos_dic_diag
        o_ref[...] = (-jnp.log(pos / neg)).reshape(1, 1)


def kernel(input_f, target, char_dic, ln1_gamma, ln1_beta):
    B, L, D = input_f.shape
    N = B * L
    n_cls = char_dic.shape[0]
    tokens = input_f.reshape(N, D)
    labels = target.reshape(N).astype(jnp.int32)
    g2 = ln1_gamma.reshape(1, D)
    b2 = ln1_beta.reshape(1, D)

    idx = jnp.arange(N, dtype=jnp.int32)
    skey = jnp.sort(labels * N + idx)
    perm = jnp.bitwise_and(skey, N - 1)
    slab = jax.lax.shift_right_logical(skey, 12)
    stok = _sc_gather(tokens, perm)

    T = 256
    nt = N // T
    lab3 = slab.reshape(nt, 1, T)

    loss = pl.pallas_call(
        functools.partial(_mega_kernel, n_cls, T, nt),
        grid=(nt + 1,),
        in_specs=[
            pl.BlockSpec((N, D), lambda p: (0, 0)),
            pl.BlockSpec((nt, 1, T), lambda p: (0, 0, 0)),
            pl.BlockSpec((n_cls, D), lambda p: (0, 0)),
            pl.BlockSpec((1, D), lambda p: (0, 0)),
            pl.BlockSpec((1, D), lambda p: (0, 0)),
        ],
        out_specs=pl.BlockSpec((1, 1), lambda p: (0, 0)),
        out_shape=jax.ShapeDtypeStruct((1, 1), jnp.float32),
        scratch_shapes=[
            pltpu.VMEM((N, D), jnp.float32),
            pltpu.VMEM((n_cls, D), jnp.float32),
            pltpu.VMEM((1, n_cls), jnp.float32),
            pltpu.SMEM((1, 1), jnp.float32),
            pltpu.SMEM((1, 1), jnp.float32),
        ],
    )(stok, lab3, char_dic, g2, b2)

    return loss.reshape(1)


# speedup vs baseline: 1.3695x; 1.0139x over previous
import functools

import jax
import jax.numpy as jnp
from jax import lax
from jax.experimental import pallas as pl
from jax.experimental.pallas import tpu as pltpu
from jax.experimental.pallas import tpu_sc as plsc

_EPS = 1e-5


def _sc_gather_body(b_per_w, nc, table_hbm, idx_hbm, out_hbm, idx_v, rows_v, sem):
    wid = lax.axis_index("s") * nc + lax.axis_index("c")
    base = wid * b_per_w
    pltpu.sync_copy(idx_hbm.at[pl.ds(base, b_per_w)], idx_v)
    pltpu.async_copy(table_hbm.at[idx_v], rows_v, sem).wait()
    pltpu.sync_copy(rows_v, out_hbm.at[pl.ds(base, b_per_w)])


def _sc_gather(table, idx):
    N, D = table.shape
    info = plsc.get_sparse_core_info()
    nc, ns = info.num_cores, info.num_subcores
    b_per_w = N // (nc * ns)
    mesh = plsc.VectorSubcoreMesh(core_axis_name="c", subcore_axis_name="s")
    return pl.kernel(
        functools.partial(_sc_gather_body, b_per_w, nc),
        mesh=mesh,
        out_type=jax.ShapeDtypeStruct((N, D), jnp.float32),
        scratch_types=[
            pltpu.VMEM((b_per_w,), jnp.int32),
            pltpu.VMEM((b_per_w, D), jnp.float32),
            pltpu.SemaphoreType.DMA,
        ],
    )(table, idx)


def _ln_rows(x, g, b):
    mu = jnp.mean(x, axis=-1, keepdims=True)
    var = jnp.mean((x - mu) ** 2, axis=-1, keepdims=True)
    return (x - mu) * jax.lax.rsqrt(var + _EPS) * g + b


def _mega_kernel(n_cls, T, nt,
                 tok_ref, lab_ref, dic_ref, g_ref, b_ref, o_ref,
                 ln_ref, seg_ref, cnt_ref, posT_ref, dpos_ref):
    p = pl.program_id(0)

    @pl.when(p == 0)
    def _init():
        seg_ref[...] = jnp.zeros_like(seg_ref)
        cnt_ref[...] = jnp.zeros_like(cnt_ref)
        posT_ref[0, 0] = 0.0
        dpos_ref[0, 0] = 0.0

    @pl.when(p < nt)
    def _band_phase():
        i = p
        row = _ln_rows(tok_ref[pl.ds(i * T, T), :], g_ref[...], b_ref[...])
        ln_ref[pl.ds(i * T, T), :] = row
        labr = lab_ref[i, 0, :]

        cls = jax.lax.broadcasted_iota(jnp.int32, (n_cls, T), 0)
        onehot = (cls == labr[None, :]).astype(jnp.float32)
        seg_ref[...] += jax.lax.dot_general(
            onehot, row, (((1,), (0,)), ((), ())),
            preferred_element_type=jnp.float32,
        )
        cnt_ref[...] += jnp.sum(onehot, axis=1)[None, :]
        g = jax.lax.dot_general(
            dic_ref[...], row, (((1,), (1,)), ((), ())),
            preferred_element_type=jnp.float32,
        ) * (1.0 / 768.0)
        dpos_ref[0, 0] += 2.0 * jnp.sum(jnp.where(onehot > 0.0, jnp.exp(g), 0.0))

        lab_all = lab_ref[...]
        row_first = jnp.min(labr)
        jlo = jnp.sum((lab_all < row_first).astype(jnp.int32)) // T

        def body(j, acc):
            col = ln_ref[pl.ds(j * T, T), :]
            s = jax.lax.dot_general(
                row, col, (((1,), (1,)), ((), ())),
                preferred_element_type=jnp.float32,
            ) * (1.0 / 768.0)
            labc = lab_ref[j, 0, :]
            m = labr[:, None] == labc[None, :]
            w = jnp.where(j == i, 1.0, 2.0)
            return acc + w * jnp.sum(jnp.where(m, jnp.exp(s), 0.0))

        acc = jax.lax.fori_loop(jlo, i + 1, body, jnp.float32(0.0))
        posT_ref[0, 0] += acc

    @pl.when(p == nt)
    def _final_phase():
        dic = dic_ref[...]
        diag = jnp.sum(dic * dic, axis=1) * (1.0 / 768.0)
        pos_dic_diag = jnp.sum(jnp.exp(diag))

        cnt = cnt_ref[0, :]
        char_tem = (dic + seg_ref[...]) / (1.0 + cnt)[:, None]
        updated = dic + 0.1 * char_tem
        rowi = jax.lax.broadcasted_iota(jnp.int32, (n_cls, 1), 0)
        new_dic = jnp.where(rowi == 0, dic, updated)
        nd = _ln_rows(new_dic, g_ref[...], b_ref[...])

        sim = jax.lax.dot_general(
            nd, nd, (((1,), (1,)), ((), ())),
            preferred_element_type=jnp.float32,
        ) * (1.0 / 768.0)
        rr = jax.lax.broadcasted_iota(jnp.int32, (n_cls, n_cls), 0)
        cc = jax.lax.broadcasted_iota(jnp.int32, (n_cls, n_cls), 1)
        keep = jnp.logical_and(rr > 0, cc > 0)
        neg = jnp.sum(jnp.where(keep, jnp.exp(sim), 0.0))

        pos = posT_ref[0, 0] + dpos_ref[0, 0] + pos_dic_diag
        o_ref[...] = (-jnp.log(pos / neg)).reshape(1, 1)


def kernel(input_f, target, char_dic, ln1_gamma, ln1_beta):
    B, L, D = input_f.shape
    N = B * L
    n_cls = char_dic.shape[0]
    tokens = input_f.reshape(N, D)
    labels = target.reshape(N).astype(jnp.int32)
    g2 = ln1_gamma.reshape(1, D)
    b2 = ln1_beta.reshape(1, D)

    idx = jnp.arange(N, dtype=jnp.int32)
    skey = jnp.sort(labels * N + idx)
    perm = jnp.bitwise_and(skey, N - 1)
    slab = jax.lax.shift_right_logical(skey, 12)
    stok = _sc_gather(tokens, perm)

    T = 256
    nt = N // T
    lab3 = slab.reshape(nt, 1, T)

    loss = pl.pallas_call(
        functools.partial(_mega_kernel, n_cls, T, nt),
        grid=(nt + 1,),
        in_specs=[
            pl.BlockSpec((N, D), lambda p: (0, 0)),
            pl.BlockSpec((nt, 1, T), lambda p: (0, 0, 0)),
            pl.BlockSpec((n_cls, D), lambda p: (0, 0)),
            pl.BlockSpec((1, D), lambda p: (0, 0)),
            pl.BlockSpec((1, D), lambda p: (0, 0)),
        ],
        out_specs=pl.BlockSpec((1, 1), lambda p: (0, 0)),
        out_shape=jax.ShapeDtypeStruct((1, 1), jnp.float32),
        scratch_shapes=[
            pltpu.VMEM((N, D), jnp.float32),
            pltpu.VMEM((n_cls, D), jnp.float32),
            pltpu.VMEM((1, n_cls), jnp.float32),
            pltpu.SMEM((1, 1), jnp.float32),
            pltpu.SMEM((1, 1), jnp.float32),
        ],
    )(stok, lab3, char_dic, g2, b2)

    return loss.reshape(1)
